# baseline (device time: 68007 ns/iter reference)
import jax
import jax.numpy as jnp
from jax import lax
from jax.experimental import pallas as pl
from jax.experimental.pallas import tpu as pltpu

N_DEV = 16
B_LOC = 2
SQ = 128
SKV = 128
D_MODEL = 512
H_LOC = 4
DH = 64
D_CHUNK = H_LOC * DH

_DD_ORDER = (0, -1, -2, -3, 1, 2, 3)


def kernel(x, Wq, K_ext, V_ext, Wo):
    pos = lax.axis_index("i")
    K_loc = lax.dynamic_slice_in_dim(K_ext, pos * B_LOC, B_LOC, axis=0)
    V_loc = lax.dynamic_slice_in_dim(V_ext, pos * B_LOC, B_LOC, axis=0)
    K_t = jnp.transpose(K_loc, (2, 0, 1, 3))
    V_t = jnp.transpose(V_loc, (2, 0, 1, 3))
    Wq = Wq.astype(jnp.bfloat16)
    Wo = Wo.astype(jnp.bfloat16)

    def body(x_ref, wq_ref, k_ref, v_ref, wo_ref, out_ref,
             comm_wq, comm_wo, recv_wq, recv_wo,
             s_up_wq, s_up_wo, s_dn_wq, s_dn_wo,
             s_cw_wq, s_cw_wo, s_ccw_wq, s_ccw_wo, s_di_wq, s_di_wo):
        my = lax.axis_index("i")
        c = lax.rem(my, 4)
        z = my // 4
        cw = 4 * z + lax.rem(c + 1, 4)
        ccw = 4 * z + lax.rem(c + 3, 4)
        up = my + 4
        dn = my - 4
        lcol = lax.rem(c + 3, 4)
        rcol = lax.rem(c + 1, 4)
        ocol = lax.rem(c + 2, 4)
        diag = 4 * z + ocol

        barrier = pltpu.get_barrier_semaphore()
        for nbr in (cw, ccw, diag, 4 * lax.rem(z + 1, 4) + c,
                    4 * lax.rem(z + 3, 4) + c):
            pl.semaphore_signal(barrier, inc=1, device_id=(nbr,),
                                device_id_type=pl.DeviceIdType.MESH)
        pl.semaphore_wait(barrier, 5)

        ti = lax.broadcasted_iota(jnp.int32, (B_LOC * SQ, B_LOC * SKV), 0)
        tj = lax.broadcasted_iota(jnp.int32, (B_LOC * SQ, B_LOC * SKV), 1)
        same_b = (ti // SQ) == (tj // SKV)
        causal = (lax.rem(tj, SKV) // 64) <= (lax.rem(ti, SQ) // 64)
        mask = jnp.logical_and(same_b, causal)

        x2 = x_ref[...].reshape(B_LOC * SQ, D_MODEL).astype(jnp.bfloat16)

        def compute_chunk(o, wq_c, wo_c):
            qs = jnp.stack(
                [jnp.dot(x2, wq_c[:, h * DH:(h + 1) * DH],
                         preferred_element_type=jnp.float32)
                 for h in range(H_LOC)], axis=0)
            k_c = k_ref[pl.ds(o * H_LOC, H_LOC)].reshape(
                H_LOC, B_LOC * SKV, DH)
            v_c = v_ref[pl.ds(o * H_LOC, H_LOC)].reshape(
                H_LOC, B_LOC * SKV, DH)
            s = lax.dot_general(
                qs, k_c, (((2,), (2,)), ((0,), (0,))),
                preferred_element_type=jnp.float32) * 0.125
            s = jnp.where(mask[None], s, -1e9)
            m = jnp.max(s, axis=2, keepdims=True)
            w = jnp.exp(s - m)
            w = w / jnp.sum(w, axis=2, keepdims=True)
            ctx = lax.dot_general(
                w, v_c, (((2,), (1,)), ((0,), (0,))),
                preferred_element_type=jnp.float32)
            wo4 = wo_c.reshape(H_LOC, DH, D_MODEL)
            outp = lax.dot_general(
                ctx.astype(jnp.bfloat16), wo4, (((2,), (1,)), ((0,), (0,))),
                preferred_element_type=jnp.float32)
            out_ref[...] += jnp.sum(outp, axis=0).reshape(
                B_LOC, SQ, D_MODEL)

        out_ref[...] = jnp.zeros_like(out_ref)

        def d_wq(o, dst, ssem, src=None):
            return pltpu.make_async_remote_copy(
                src_ref=comm_wq.at[o] if src is None else src,
                dst_ref=comm_wq.at[o],
                send_sem=ssem, recv_sem=recv_wq.at[o],
                device_id=(dst,), device_id_type=pl.DeviceIdType.MESH)

        def d_wo(o, dst, ssem, src=None):
            return pltpu.make_async_remote_copy(
                src_ref=comm_wo.at[o] if src is None else src,
                dst_ref=comm_wo.at[o],
                send_sem=ssem, recv_sem=recv_wo.at[o],
                device_id=(dst,), device_id_type=pl.DeviceIdType.MESH)

        @pl.when(z < 3)
        def _():
            d_wq(my, up, s_up_wq.at[z], src=wq_ref).start()
            d_wo(my, up, s_up_wo.at[z], src=wo_ref).start()

        @pl.when(z > 0)
        def _():
            d_wq(my, dn, s_dn_wq.at[z], src=wq_ref).start()
            d_wo(my, dn, s_dn_wo.at[z], src=wo_ref).start()

        d_wq(my, cw, s_cw_wq.at[z], src=wq_ref).start()
        d_wo(my, cw, s_cw_wo.at[z], src=wo_ref).start()
        d_wq(my, ccw, s_ccw_wq.at[z], src=wq_ref).start()
        d_wo(my, ccw, s_ccw_wo.at[z], src=wo_ref).start()
        d_wq(my, diag, s_di_wq.at[z], src=wq_ref).start()
        d_wo(my, diag, s_di_wo.at[z], src=wo_ref).start()

        compute_chunk(my, wq_ref[...], wo_ref[...])

        for d in range(1, 4):
            @pl.when(z >= d)
            def _(d=d):
                zp = z - d
                o = my - 4 * d

                d_wq(o, up, s_up_wq.at[zp]).wait_recv()

                @pl.when(z < 3)
                def _():
                    d_wq(o, up, s_up_wq.at[zp]).start()

                d_wq(o, cw, s_cw_wq.at[zp]).start()
                d_wq(o, ccw, s_ccw_wq.at[zp]).start()
                d_wq(o, diag, s_di_wq.at[zp]).start()

                d_wo(o, up, s_up_wo.at[zp]).wait_recv()

                @pl.when(z < 3)
                def _():
                    d_wo(o, up, s_up_wo.at[zp]).start()

                d_wo(o, cw, s_cw_wo.at[zp]).start()
                d_wo(o, ccw, s_ccw_wo.at[zp]).start()
                d_wo(o, diag, s_di_wo.at[zp]).start()
                compute_chunk(o, comm_wq[o], comm_wo[o])

            @pl.when(z + d <= 3)
            def _(d=d):
                zp = z + d
                o = my + 4 * d

                d_wq(o, dn, s_dn_wq.at[zp]).wait_recv()

                @pl.when(z > 0)
                def _():
                    d_wq(o, dn, s_dn_wq.at[zp]).start()

                d_wq(o, cw, s_cw_wq.at[zp]).start()
                d_wq(o, ccw, s_ccw_wq.at[zp]).start()
                d_wq(o, diag, s_di_wq.at[zp]).start()

                d_wo(o, dn, s_dn_wo.at[zp]).wait_recv()

                @pl.when(z > 0)
                def _():
                    d_wo(o, dn, s_dn_wo.at[zp]).start()

                d_wo(o, cw, s_cw_wo.at[zp]).start()
                d_wo(o, ccw, s_ccw_wo.at[zp]).start()
                d_wo(o, diag, s_di_wo.at[zp]).start()
                compute_chunk(o, comm_wq[o], comm_wo[o])

        for dd in _DD_ORDER:
            @pl.when(jnp.logical_and(z + dd >= 0, z + dd <= 3))
            def _(dd=dd):
                zp = z + dd
                oL = 4 * zp + lcol
                oR = 4 * zp + rcol
                d_wq(oL, cw, s_cw_wq.at[zp]).wait_recv()
                d_wo(oL, cw, s_cw_wo.at[zp]).wait_recv()
                d_wq(oR, cw, s_cw_wq.at[zp]).wait_recv()
                d_wo(oR, cw, s_cw_wo.at[zp]).wait_recv()
                compute_chunk(oL, comm_wq[oL], comm_wo[oL])
                compute_chunk(oR, comm_wq[oR], comm_wo[oR])

        for dd in _DD_ORDER:
            @pl.when(jnp.logical_and(z + dd >= 0, z + dd <= 3))
            def _(dd=dd):
                zp = z + dd
                o = 4 * zp + ocol
                d_wq(o, cw, s_cw_wq.at[zp]).wait_recv()
                d_wo(o, cw, s_cw_wo.at[zp]).wait_recv()
                compute_chunk(o, comm_wq[o], comm_wo[o])

        for zp in range(4):
            @pl.when(jnp.logical_and(z < 3, zp <= z))
            def _(zp=zp):
                d_wq(0, cw, s_up_wq.at[zp]).wait_send()
                d_wo(0, cw, s_up_wo.at[zp]).wait_send()

            @pl.when(jnp.logical_and(z > 0, zp >= z))
            def _(zp=zp):
                d_wq(0, cw, s_dn_wq.at[zp]).wait_send()
                d_wo(0, cw, s_dn_wo.at[zp]).wait_send()

            d_wq(0, cw, s_cw_wq.at[zp]).wait_send()
            d_wo(0, cw, s_cw_wo.at[zp]).wait_send()
            d_wq(0, cw, s_ccw_wq.at[zp]).wait_send()
            d_wo(0, cw, s_ccw_wo.at[zp]).wait_send()
            d_wq(0, cw, s_di_wq.at[zp]).wait_send()
            d_wo(0, cw, s_di_wo.at[zp]).wait_send()

    return pl.pallas_call(
        body,
        out_shape=jax.ShapeDtypeStruct((B_LOC, SQ, D_MODEL), jnp.float32),
        in_specs=[pl.BlockSpec(memory_space=pltpu.VMEM)] * 5,
        out_specs=pl.BlockSpec(memory_space=pltpu.VMEM),
        scratch_shapes=[
            pltpu.VMEM((N_DEV, D_MODEL, D_CHUNK), jnp.bfloat16),
            pltpu.VMEM((N_DEV, D_CHUNK, D_MODEL), jnp.bfloat16),
            pltpu.SemaphoreType.DMA((N_DEV,)),
            pltpu.SemaphoreType.DMA((N_DEV,)),
            pltpu.SemaphoreType.DMA((4,)),
            pltpu.SemaphoreType.DMA((4,)),
            pltpu.SemaphoreType.DMA((4,)),
            pltpu.SemaphoreType.DMA((4,)),
            pltpu.SemaphoreType.DMA((4,)),
            pltpu.SemaphoreType.DMA((4,)),
            pltpu.SemaphoreType.DMA((4,)),
            pltpu.SemaphoreType.DMA((4,)),
            pltpu.SemaphoreType.DMA((4,)),
            pltpu.SemaphoreType.DMA((4,)),
        ],
        compiler_params=pltpu.CompilerParams(collective_id=0),
    )(x, Wq, K_t, V_t, Wo)


# device time: 55026 ns/iter; 1.2359x vs baseline; 1.2359x over previous
import jax
import jax.numpy as jnp
from jax import lax
from jax.experimental import pallas as pl
from jax.experimental.pallas import tpu as pltpu

N_DEV = 16
B_LOC = 2
SQ = 128
SKV = 128
D_MODEL = 512
H_LOC = 4
DH = 64
D_CHUNK = H_LOC * DH

_DD_ORDER = (0, -1, -2, -3, 1, 2, 3)


def kernel(x, Wq, K_ext, V_ext, Wo):
    pos = lax.axis_index("i")
    K_loc = lax.dynamic_slice_in_dim(K_ext, pos * B_LOC, B_LOC, axis=0)
    V_loc = lax.dynamic_slice_in_dim(V_ext, pos * B_LOC, B_LOC, axis=0)
    K_t = jnp.transpose(K_loc, (2, 0, 1, 3))
    V_t = jnp.transpose(V_loc, (2, 0, 1, 3))
    Wq = Wq.astype(jnp.bfloat16)
    Wo = Wo.astype(jnp.bfloat16)

    def body(x_ref, wq_ref, k_ref, v_ref, wo_ref, out_ref,
             comm_wq, comm_wo, recv_wq, recv_wo,
             s_up_wq, s_up_wo, s_dn_wq, s_dn_wo,
             s_cw_wq, s_cw_wo, s_ccw_wq, s_ccw_wo, s_fq, s_fo):
        my = lax.axis_index("i")
        c = lax.rem(my, 4)
        z = my // 4
        cw = 4 * z + lax.rem(c + 1, 4)
        ccw = 4 * z + lax.rem(c + 3, 4)
        up = my + 4
        dn = my - 4
        lcol = lax.rem(c + 3, 4)
        rcol = lax.rem(c + 1, 4)
        ocol = lax.rem(c + 2, 4)

        barrier = pltpu.get_barrier_semaphore()
        for nbr in (cw, ccw, 4 * lax.rem(z + 1, 4) + c,
                    4 * lax.rem(z + 3, 4) + c):
            pl.semaphore_signal(barrier, inc=1, device_id=(nbr,),
                                device_id_type=pl.DeviceIdType.MESH)
        pl.semaphore_wait(barrier, 4)

        ti = lax.broadcasted_iota(jnp.int32, (B_LOC * SQ, B_LOC * SKV), 0)
        tj = lax.broadcasted_iota(jnp.int32, (B_LOC * SQ, B_LOC * SKV), 1)
        same_b = (ti // SQ) == (tj // SKV)
        causal = (lax.rem(tj, SKV) // 64) <= (lax.rem(ti, SQ) // 64)
        mask = jnp.logical_and(same_b, causal)

        x2 = x_ref[...].reshape(B_LOC * SQ, D_MODEL).astype(jnp.bfloat16)

        def compute_chunk(o, wq_c, wo_c):
            qs = jnp.stack(
                [jnp.dot(x2, wq_c[:, h * DH:(h + 1) * DH],
                         preferred_element_type=jnp.float32)
                 for h in range(H_LOC)], axis=0)
            k_c = k_ref[pl.ds(o * H_LOC, H_LOC)].reshape(
                H_LOC, B_LOC * SKV, DH)
            v_c = v_ref[pl.ds(o * H_LOC, H_LOC)].reshape(
                H_LOC, B_LOC * SKV, DH)
            s = lax.dot_general(
                qs, k_c, (((2,), (2,)), ((0,), (0,))),
                preferred_element_type=jnp.float32) * 0.125
            s = jnp.where(mask[None], s, -1e9)
            m = jnp.max(s, axis=2, keepdims=True)
            w = jnp.exp(s - m)
            w = w / jnp.sum(w, axis=2, keepdims=True)
            ctx = lax.dot_general(
                w, v_c, (((2,), (1,)), ((0,), (0,))),
                preferred_element_type=jnp.float32)
            wo4 = wo_c.reshape(H_LOC, DH, D_MODEL)
            outp = lax.dot_general(
                ctx.astype(jnp.bfloat16), wo4, (((2,), (1,)), ((0,), (0,))),
                preferred_element_type=jnp.float32)
            out_ref[...] += jnp.sum(outp, axis=0).reshape(
                B_LOC, SQ, D_MODEL)

        out_ref[...] = jnp.zeros_like(out_ref)

        def d_wq(o, dst, ssem, src=None):
            return pltpu.make_async_remote_copy(
                src_ref=comm_wq.at[o] if src is None else src,
                dst_ref=comm_wq.at[o],
                send_sem=ssem, recv_sem=recv_wq.at[o],
                device_id=(dst,), device_id_type=pl.DeviceIdType.MESH)

        def d_wo(o, dst, ssem, src=None):
            return pltpu.make_async_remote_copy(
                src_ref=comm_wo.at[o] if src is None else src,
                dst_ref=comm_wo.at[o],
                send_sem=ssem, recv_sem=recv_wo.at[o],
                device_id=(dst,), device_id_type=pl.DeviceIdType.MESH)

        @pl.when(z < 3)
        def _():
            d_wq(my, up, s_up_wq.at[z], src=wq_ref).start()
            d_wo(my, up, s_up_wo.at[z], src=wo_ref).start()

        @pl.when(z > 0)
        def _():
            d_wq(my, dn, s_dn_wq.at[z], src=wq_ref).start()
            d_wo(my, dn, s_dn_wo.at[z], src=wo_ref).start()

        d_wq(my, cw, s_cw_wq.at[z], src=wq_ref).start()
        d_wo(my, cw, s_cw_wo.at[z], src=wo_ref).start()
        d_wq(my, ccw, s_ccw_wq.at[z], src=wq_ref).start()
        d_wo(my, ccw, s_ccw_wo.at[z], src=wo_ref).start()

        compute_chunk(my, wq_ref[...], wo_ref[...])

        for d in range(1, 4):
            @pl.when(z >= d)
            def _(d=d):
                zp = z - d
                o = my - 4 * d

                d_wq(o, up, s_up_wq.at[zp]).wait_recv()

                @pl.when(z < 3)
                def _():
                    d_wq(o, up, s_up_wq.at[zp]).start()

                d_wq(o, cw, s_cw_wq.at[zp]).start()
                d_wq(o, ccw, s_ccw_wq.at[zp]).start()

                d_wo(o, up, s_up_wo.at[zp]).wait_recv()

                @pl.when(z < 3)
                def _():
                    d_wo(o, up, s_up_wo.at[zp]).start()

                d_wo(o, cw, s_cw_wo.at[zp]).start()
                d_wo(o, ccw, s_ccw_wo.at[zp]).start()
                compute_chunk(o, comm_wq[o], comm_wo[o])

            @pl.when(z + d <= 3)
            def _(d=d):
                zp = z + d
                o = my + 4 * d

                d_wq(o, dn, s_dn_wq.at[zp]).wait_recv()

                @pl.when(z > 0)
                def _():
                    d_wq(o, dn, s_dn_wq.at[zp]).start()

                d_wq(o, cw, s_cw_wq.at[zp]).start()
                d_wq(o, ccw, s_ccw_wq.at[zp]).start()

                d_wo(o, dn, s_dn_wo.at[zp]).wait_recv()

                @pl.when(z > 0)
                def _():
                    d_wo(o, dn, s_dn_wo.at[zp]).start()

                d_wo(o, cw, s_cw_wo.at[zp]).start()
                d_wo(o, ccw, s_ccw_wo.at[zp]).start()
                compute_chunk(o, comm_wq[o], comm_wo[o])

        for dd in _DD_ORDER:
            @pl.when(jnp.logical_and(z + dd >= 0, z + dd <= 3))
            def _(dd=dd):
                zp = z + dd
                oL = 4 * zp + lcol
                oR = 4 * zp + rcol
                d_wq(oL, cw, s_fq.at[zp]).wait_recv()
                d_wq(oL, cw, s_fq.at[zp]).start()
                d_wq(oR, cw, s_fq.at[zp]).wait_recv()
                d_wo(oL, ccw, s_fo.at[zp]).wait_recv()
                d_wo(oR, ccw, s_fo.at[zp]).wait_recv()
                d_wo(oR, ccw, s_fo.at[zp]).start()
                compute_chunk(oL, comm_wq[oL], comm_wo[oL])
                compute_chunk(oR, comm_wq[oR], comm_wo[oR])

        for dd in _DD_ORDER:
            @pl.when(jnp.logical_and(z + dd >= 0, z + dd <= 3))
            def _(dd=dd):
                zp = z + dd
                o = 4 * zp + ocol
                d_wq(o, cw, s_fq.at[zp]).wait_recv()
                d_wo(o, ccw, s_fo.at[zp]).wait_recv()
                compute_chunk(o, comm_wq[o], comm_wo[o])

        for zp in range(4):
            @pl.when(jnp.logical_and(z < 3, zp <= z))
            def _(zp=zp):
                d_wq(0, cw, s_up_wq.at[zp]).wait_send()
                d_wo(0, cw, s_up_wo.at[zp]).wait_send()

            @pl.when(jnp.logical_and(z > 0, zp >= z))
            def _(zp=zp):
                d_wq(0, cw, s_dn_wq.at[zp]).wait_send()
                d_wo(0, cw, s_dn_wo.at[zp]).wait_send()

            d_wq(0, cw, s_cw_wq.at[zp]).wait_send()
            d_wo(0, cw, s_cw_wo.at[zp]).wait_send()
            d_wq(0, cw, s_ccw_wq.at[zp]).wait_send()
            d_wo(0, cw, s_ccw_wo.at[zp]).wait_send()
            d_wq(0, cw, s_fq.at[zp]).wait_send()
            d_wo(0, cw, s_fo.at[zp]).wait_send()

    return pl.pallas_call(
        body,
        out_shape=jax.ShapeDtypeStruct((B_LOC, SQ, D_MODEL), jnp.float32),
        in_specs=[pl.BlockSpec(memory_space=pltpu.VMEM)] * 5,
        out_specs=pl.BlockSpec(memory_space=pltpu.VMEM),
        scratch_shapes=[
            pltpu.VMEM((N_DEV, D_MODEL, D_CHUNK), jnp.bfloat16),
            pltpu.VMEM((N_DEV, D_CHUNK, D_MODEL), jnp.bfloat16),
            pltpu.SemaphoreType.DMA((N_DEV,)),
            pltpu.SemaphoreType.DMA((N_DEV,)),
            pltpu.SemaphoreType.DMA((4,)),
            pltpu.SemaphoreType.DMA((4,)),
            pltpu.SemaphoreType.DMA((4,)),
            pltpu.SemaphoreType.DMA((4,)),
            pltpu.SemaphoreType.DMA((4,)),
            pltpu.SemaphoreType.DMA((4,)),
            pltpu.SemaphoreType.DMA((4,)),
            pltpu.SemaphoreType.DMA((4,)),
            pltpu.SemaphoreType.DMA((4,)),
            pltpu.SemaphoreType.DMA((4,)),
        ],
        compiler_params=pltpu.CompilerParams(collective_id=0),
    )(x, Wq, K_t, V_t, Wo)
